# Initial kernel scaffold; baseline (speedup 1.0000x reference)
import jax
import jax.numpy as jnp
from jax.experimental import pallas as pl


def kernel(f, g, edge_index, params):
    raise NotImplementedError("write your pallas kernel here")


# speedup vs baseline: 6.3116x; 6.3116x over previous
import functools

import jax
import jax.numpy as jnp
from jax import lax
from jax.experimental import pallas as pl
from jax.experimental.pallas import tpu as pltpu
from jax.experimental.pallas import tpu_sc as plsc

_N = 100000
_NC = 32
_E = 1600000
_HALF = 50000
_ACC_ROWS = 50304
_EP_ROWS = 12544
_R_SUB = 784
_C = 4
_ITERS = _R_SUB // _C
_ZROWS = _ACC_ROWS // 16


def _shift_body(x_hbm, src_hbm, dst_hbm, zeros_hbm, out_hbm,
                acc, src_v, dst_v, rows_v, gsem):
    c = lax.axis_index("c")
    s = lax.axis_index("s")
    lo = c * _HALF
    hi = lo + _HALF
    gvec = jnp.arange(16, dtype=jnp.int32) + (_HALF + s * 16)

    pltpu.sync_copy(zeros_hbm, rows_v)
    z0 = s * _ZROWS
    for k in range(6):
        pltpu.sync_copy(rows_v, acc.at[pl.ds(z0 + k * 512, 512)])
    pltpu.sync_copy(rows_v.at[pl.ds(0, 72)], acc.at[pl.ds(z0 + 3072, 72)])
    plsc.subcore_barrier()

    base_row = s * _R_SUB

    def _edge_iter(it, carry):
        row = base_row + it * _C
        pltpu.sync_copy(src_hbm.at[pl.ds(row, _C)], src_v)
        pltpu.sync_copy(dst_hbm.at[pl.ds(row, _C)], dst_v)
        copies = [
            pltpu.async_copy(x_hbm.at[src_v.at[j]],
                             rows_v.at[pl.ds(j * 128, 128)], gsem)
            for j in range(_C)
        ]
        for j in range(_C):
            for v in range(8):
                d = dst_v[j, pl.ds(v * 16, 16)]
                inr = (d >= lo) & (d < hi)
                dst_v[j, pl.ds(v * 16, 16)] = jnp.where(inr, d - lo, gvec)
        for cp in copies:
            cp.wait()
        for j in range(_C):
            pltpu.sync_copy(rows_v.at[pl.ds(j * 128, 128)],
                            acc.at[dst_v.at[j]], add=True)
        return carry

    lax.fori_loop(0, _ITERS, _edge_iter, 0)
    plsc.subcore_barrier()

    a0 = s * 3128

    @pl.when(s < 15)
    def _wb_main():
        pltpu.sync_copy(acc.at[pl.ds(a0, 3128)],
                        out_hbm.at[pl.ds(c * _HALF + a0, 3128)])

    @pl.when(s == 15)
    def _wb_tail():
        pltpu.sync_copy(acc.at[pl.ds(a0, 3080)],
                        out_hbm.at[pl.ds(c * _HALF + a0, 3080)])


@functools.cache
def _shift_call():
    return pl.kernel(
        _shift_body,
        out_type=jax.ShapeDtypeStruct((_N, _NC), jnp.float32),
        mesh=plsc.VectorSubcoreMesh(core_axis_name="c", subcore_axis_name="s"),
        compiler_params=pltpu.CompilerParams(use_tc_tiling_on_sc=False),
        scratch_types=[
            pltpu.VMEM_SHARED((_ACC_ROWS, _NC), jnp.float32),
            pltpu.VMEM((_C, 128), jnp.int32),
            pltpu.VMEM((_C, 128), jnp.int32),
            pltpu.VMEM((_C * 128, _NC), jnp.float32),
            pltpu.SemaphoreType.DMA,
        ],
    )


def _mlp_kernel(x_ref, w1_ref, b1_ref, w2_ref, b2_ref, o_ref):
    h = jnp.dot(x_ref[...], w1_ref[...], preferred_element_type=jnp.float32)
    h = jnp.maximum(h + b1_ref[...], 0.0)
    o_ref[...] = jnp.dot(h, w2_ref[...],
                         preferred_element_type=jnp.float32) + b2_ref[...]


def _mlp_res_kernel(x_ref, r_ref, w1_ref, b1_ref, w2_ref, b2_ref, o_ref):
    h = jnp.dot(x_ref[...], w1_ref[...], preferred_element_type=jnp.float32)
    h = jnp.maximum(h + b1_ref[...], 0.0)
    o_ref[...] = (r_ref[...] + jnp.dot(h, w2_ref[...],
                                       preferred_element_type=jnp.float32)
                  + b2_ref[...])


_ROW_BLK = 2000


def _mlp(x, p, residual=None):
    n, d_in = x.shape
    hdim = p["W1"].shape[1]
    d_out = p["W2"].shape[1]
    b1 = p["b1"].reshape(1, hdim)
    b2 = p["b2"].reshape(1, d_out)
    grid = (n // _ROW_BLK,)
    x_spec = pl.BlockSpec((_ROW_BLK, d_in), lambda i: (i, 0))
    w1_spec = pl.BlockSpec((d_in, hdim), lambda i: (0, 0))
    b1_spec = pl.BlockSpec((1, hdim), lambda i: (0, 0))
    w2_spec = pl.BlockSpec((hdim, d_out), lambda i: (0, 0))
    b2_spec = pl.BlockSpec((1, d_out), lambda i: (0, 0))
    o_spec = pl.BlockSpec((_ROW_BLK, d_out), lambda i: (i, 0))
    out_shape = jax.ShapeDtypeStruct((n, d_out), jnp.float32)
    if residual is None:
        return pl.pallas_call(
            _mlp_kernel, grid=grid,
            in_specs=[x_spec, w1_spec, b1_spec, w2_spec, b2_spec],
            out_specs=o_spec, out_shape=out_shape,
        )(x, p["W1"], b1, p["W2"], b2)
    r_spec = pl.BlockSpec((_ROW_BLK, d_out), lambda i: (i, 0))
    return pl.pallas_call(
        _mlp_res_kernel, grid=grid,
        in_specs=[x_spec, r_spec, w1_spec, b1_spec, w2_spec, b2_spec],
        out_specs=o_spec, out_shape=out_shape,
    )(x, residual, p["W1"], b1, p["W2"], b2)


def kernel(f, g, edge_index, params):
    src = edge_index[0]
    dst = edge_index[1]
    pad = _EP_ROWS * 128 - _E
    src_p = jnp.concatenate(
        [src, jnp.zeros((pad,), jnp.int32)]).reshape(_EP_ROWS, 128)
    dst_p = jnp.concatenate(
        [dst, jnp.full((pad,), _N, jnp.int32)]).reshape(_EP_ROWS, 128)
    zeros_stage = jnp.zeros((_C * 128, _NC), jnp.float32)

    f1 = _mlp(f, params["readin_f"])
    g1 = _mlp(g, params["readin_g"])
    for l in range(2):
        fp = _shift_call()(f1, src_p, dst_p, zeros_stage)
        gp = _shift_call()(g1, src_p, dst_p, zeros_stage)
        f1n = _mlp(gp, params["convs"][l]["equi"], residual=f1)
        g1n = _mlp(fp, params["convs"][l]["inv"], residual=g1)
        f1, g1 = f1n, g1n
    return (_mlp(f1, params["readout_f"]), _mlp(g1, params["readout_g"]))


# speedup vs baseline: 8.3163x; 1.3176x over previous
import functools

import jax
import jax.numpy as jnp
from jax import lax
from jax.experimental import pallas as pl
from jax.experimental.pallas import tpu as pltpu
from jax.experimental.pallas import tpu_sc as plsc

_N = 100000
_NC = 32
_E = 1600000
_HALF = 50000
_ACC_ROWS = 50048
_EP_ROWS = 12672
_R_SUB = 792
_C = 12
_ITERS = _R_SUB // _C
_ZROWS = _ACC_ROWS // 16


def _shift_body(x_hbm, src_hbm, dst_hbm, zeros_hbm, out_hbm,
                acc, src_v, dst_v, rows_v, gsem):
    c = lax.axis_index("c")
    s = lax.axis_index("s")
    lo = c * _HALF
    hi = lo + _HALF
    gvec = jnp.arange(16, dtype=jnp.int32) + (_HALF + 16 * (s % 3))

    pltpu.sync_copy(zeros_hbm, rows_v)
    z0 = s * _ZROWS
    for k in range(2):
        pltpu.sync_copy(rows_v, acc.at[pl.ds(z0 + k * 1536, 1536)])
    pltpu.sync_copy(rows_v.at[pl.ds(0, 56)], acc.at[pl.ds(z0 + 3072, 56)])
    plsc.subcore_barrier()

    base_row = s * _R_SUB

    def _edge_iter(it, carry):
        row = base_row + it * _C
        pltpu.sync_copy(src_hbm.at[pl.ds(row, _C)], src_v)
        pltpu.sync_copy(dst_hbm.at[pl.ds(row, _C)], dst_v)
        copies = [
            pltpu.async_copy(x_hbm.at[src_v.at[j]],
                             rows_v.at[pl.ds(j * 128, 128)], gsem)
            for j in range(_C)
        ]
        for j in range(_C):
            for v in range(8):
                d = dst_v[j, pl.ds(v * 16, 16)]
                inr = (d >= lo) & (d < hi)
                dst_v[j, pl.ds(v * 16, 16)] = jnp.where(inr, d - lo, gvec)
        for cp in copies:
            cp.wait()
        adds = [
            pltpu.async_copy(rows_v.at[pl.ds(j * 128, 128)],
                             acc.at[dst_v.at[j]], gsem, add=True)
            for j in range(_C)
        ]
        for ad in adds:
            ad.wait()
        return carry

    lax.fori_loop(0, _ITERS, _edge_iter, 0)
    plsc.subcore_barrier()

    a0 = s * 3128

    @pl.when(s < 15)
    def _wb_main():
        pltpu.sync_copy(acc.at[pl.ds(a0, 3128)],
                        out_hbm.at[pl.ds(c * _HALF + a0, 3128)])

    @pl.when(s == 15)
    def _wb_tail():
        pltpu.sync_copy(acc.at[pl.ds(a0, 3080)],
                        out_hbm.at[pl.ds(c * _HALF + a0, 3080)])


@functools.cache
def _shift_call():
    return pl.kernel(
        _shift_body,
        out_type=jax.ShapeDtypeStruct((_N, _NC), jnp.bfloat16),
        mesh=plsc.VectorSubcoreMesh(core_axis_name="c", subcore_axis_name="s"),
        compiler_params=pltpu.CompilerParams(use_tc_tiling_on_sc=False),
        scratch_types=[
            pltpu.VMEM_SHARED((_ACC_ROWS, _NC), jnp.bfloat16),
            pltpu.VMEM((_C, 128), jnp.int32),
            pltpu.VMEM((_C, 128), jnp.int32),
            pltpu.VMEM((_C * 128, _NC), jnp.bfloat16),
            pltpu.SemaphoreType.DMA,
        ],
    )


def _mlp_kernel(x_ref, w1_ref, b1_ref, w2_ref, b2_ref, o_ref):
    x = x_ref[...].astype(jnp.float32)
    h = jnp.dot(x, w1_ref[...], preferred_element_type=jnp.float32)
    h = jnp.maximum(h + b1_ref[...], 0.0)
    o_ref[...] = jnp.dot(h, w2_ref[...],
                         preferred_element_type=jnp.float32) + b2_ref[...]


def _mlp_res_kernel(x_ref, r_ref, w1_ref, b1_ref, w2_ref, b2_ref, o_ref):
    x = x_ref[...].astype(jnp.float32)
    h = jnp.dot(x, w1_ref[...], preferred_element_type=jnp.float32)
    h = jnp.maximum(h + b1_ref[...], 0.0)
    o_ref[...] = (r_ref[...] + jnp.dot(h, w2_ref[...],
                                       preferred_element_type=jnp.float32)
                  + b2_ref[...])


_ROW_BLK = 2000


def _mlp(x, p, residual=None):
    n, d_in = x.shape
    hdim = p["W1"].shape[1]
    d_out = p["W2"].shape[1]
    b1 = p["b1"].reshape(1, hdim)
    b2 = p["b2"].reshape(1, d_out)
    grid = (n // _ROW_BLK,)
    x_spec = pl.BlockSpec((_ROW_BLK, d_in), lambda i: (i, 0))
    w1_spec = pl.BlockSpec((d_in, hdim), lambda i: (0, 0))
    b1_spec = pl.BlockSpec((1, hdim), lambda i: (0, 0))
    w2_spec = pl.BlockSpec((hdim, d_out), lambda i: (0, 0))
    b2_spec = pl.BlockSpec((1, d_out), lambda i: (0, 0))
    o_spec = pl.BlockSpec((_ROW_BLK, d_out), lambda i: (i, 0))
    out_shape = jax.ShapeDtypeStruct((n, d_out), jnp.float32)
    if residual is None:
        return pl.pallas_call(
            _mlp_kernel, grid=grid,
            in_specs=[x_spec, w1_spec, b1_spec, w2_spec, b2_spec],
            out_specs=o_spec, out_shape=out_shape,
        )(x, p["W1"], b1, p["W2"], b2)
    r_spec = pl.BlockSpec((_ROW_BLK, d_out), lambda i: (i, 0))
    return pl.pallas_call(
        _mlp_res_kernel, grid=grid,
        in_specs=[x_spec, r_spec, w1_spec, b1_spec, w2_spec, b2_spec],
        out_specs=o_spec, out_shape=out_shape,
    )(x, residual, p["W1"], b1, p["W2"], b2)


def kernel(f, g, edge_index, params):
    src = edge_index[0]
    dst = edge_index[1]
    pad = _EP_ROWS * 128 - _E
    src_p = jnp.concatenate(
        [src, jnp.zeros((pad,), jnp.int32)]).reshape(_EP_ROWS, 128)
    dst_p = jnp.concatenate(
        [dst, jnp.full((pad,), _N, jnp.int32)]).reshape(_EP_ROWS, 128)
    zeros_stage = jnp.zeros((_C * 128, _NC), jnp.bfloat16)

    f1 = _mlp(f, params["readin_f"])
    g1 = _mlp(g, params["readin_g"])
    for l in range(2):
        fp = _shift_call()(f1.astype(jnp.bfloat16), src_p, dst_p, zeros_stage)
        gp = _shift_call()(g1.astype(jnp.bfloat16), src_p, dst_p, zeros_stage)
        f1n = _mlp(gp, params["convs"][l]["equi"], residual=f1)
        g1n = _mlp(fp, params["convs"][l]["inv"], residual=g1)
        f1, g1 = f1n, g1n
    return (_mlp(f1, params["readout_f"]), _mlp(g1, params["readout_g"]))
